# Initial kernel scaffold; baseline (speedup 1.0000x reference)
import jax
import jax.numpy as jnp
from jax.experimental import pallas as pl


def kernel(atom_type_array, embedding_table):
    raise NotImplementedError("write your pallas kernel here")


# speedup vs baseline: 3.6950x; 3.6950x over previous
import functools

import jax
import jax.numpy as jnp
from jax import lax
from jax.experimental import pallas as pl
from jax.experimental.pallas import tpu as pltpu
from jax.experimental.pallas import tpu_sc as plsc

NUM_ELEMENTS = 100
EMBED_DIM = 128

_B = 16384 * 200
_NC = 2
_NS = 16
_NW = _NC * _NS
_BPW = _B // _NW
_K = 8
_GROUP = _K * 128
_HALF = _GROUP // 2
_NGROUP = _BPW // _GROUP
_IDX_ROWS_PER_W = _BPW // 128


def _make_sc_kernel():
    mesh = plsc.VectorSubcoreMesh(core_axis_name="c", subcore_axis_name="s")

    @functools.partial(
        pl.kernel,
        mesh=mesh,
        out_type=jax.ShapeDtypeStruct((_B, EMBED_DIM), jnp.float32),
        scratch_types=[
            pltpu.VMEM((_K, 128), jnp.int32),
            pltpu.VMEM((_HALF, EMBED_DIM), jnp.float32),
            pltpu.SemaphoreType.DMA,
        ],
    )
    def emb(table_hbm, idx_hbm, out_hbm, idx_v, rows_v, sem):
        wid = lax.axis_index("s") * _NC + lax.axis_index("c")
        idx_row_base = wid * _IDX_ROWS_PER_W
        out_base = wid * _BPW

        def group_body(i, carry):
            pltpu.sync_copy(idx_hbm.at[pl.ds(idx_row_base + i * _K, _K)], idx_v)
            for h in range(2):
                descs = [
                    pltpu.async_copy(
                        table_hbm.at[idx_v.at[h * (_K // 2) + j]],
                        rows_v.at[pl.ds(j * 128, 128)],
                        sem,
                    )
                    for j in range(_K // 2)
                ]
                for d in descs:
                    d.wait()
                pltpu.sync_copy(
                    rows_v,
                    out_hbm.at[pl.ds(out_base + i * _GROUP + h * _HALF, _HALF)],
                )
            return carry

        lax.fori_loop(0, _NGROUP, group_body, 0)

    return emb


_emb_kernel = _make_sc_kernel()


@jax.jit
def kernel(atom_type_array, embedding_table):
    idx2d = atom_type_array.astype(jnp.int32).reshape(_B // 128, 128)
    out = _emb_kernel(embedding_table, idx2d)
    return out.reshape(atom_type_array.shape + (EMBED_DIM,))
